# baseline (speedup 1.0000x reference)
import jax
import jax.numpy as jnp
from jax import lax
from jax.experimental import pallas as pl
from jax.experimental.pallas import tpu as pltpu

PI = 3.141592653589793
EPS = 1e-7

B, A, NC = 16, 21504, 80
AB2 = 5376
GRID2 = A // AB2


def _acos(x):
    ax = jnp.abs(x)
    p = (1.5707963050 + ax * (-0.2145988016 + ax * (0.0889789874 + ax * (
        -0.0501743046 + ax * (0.0308918810 + ax * (-0.0170881256 + ax * (
            0.0066700901 + ax * -0.0012624911)))))))
    r = jnp.sqrt(jnp.maximum(1.0 - ax, 0.0)) * p
    return jnp.where(x >= 0.0, r, PI - r)


def _circle_losses(x1, y1, r1, x2, y2, r2):
    d2 = (x1 - x2) ** 2 + (y1 - y2) ** 2
    d = jnp.sqrt(jnp.maximum(d2, EPS))
    rsum = r1 + r2
    rdiff = jnp.abs(r1 - r2)
    rmin = jnp.minimum(r1, r2)
    no_overlap = d >= rsum
    contained = d <= rdiff
    a1 = jnp.clip((d2 + r1 ** 2 - r2 ** 2) / (2.0 * d * jnp.maximum(r1, EPS)),
                  -1.0 + 1e-6, 1.0 - 1e-6)
    a2 = jnp.clip((d2 + r2 ** 2 - r1 ** 2) / (2.0 * d * jnp.maximum(r2, EPS)),
                  -1.0 + 1e-6, 1.0 - 1e-6)
    tri = jnp.maximum((-d + rsum) * (d + r1 - r2) * (d - r1 + r2) * (d + rsum),
                      EPS)
    lens = (r1 ** 2 * _acos(a1) + r2 ** 2 * _acos(a2)
            - 0.5 * jnp.sqrt(tri))
    inter = jnp.where(no_overlap, 0.0, jnp.where(contained, PI * rmin ** 2, lens))
    union = PI * (r1 ** 2 + r2 ** 2) - inter
    iou = inter / (union + EPS)
    dist = jnp.clip(1.0 - d / (rsum + EPS), 0.0, 1.0)
    return iou, dist


def _wsum_body(s_ref, w_ref):
    w_ref[...] = jnp.sum(s_ref[...], axis=0, keepdims=True)[None]


@jax.jit
def _wsum(st2):
    w3 = pl.pallas_call(
        _wsum_body,
        grid=(B,),
        in_specs=[pl.BlockSpec((NC, A), lambda i: (i, 0))],
        out_specs=pl.BlockSpec((1, 1, A), lambda i: (i, 0, 0)),
        out_shape=jax.ShapeDtypeStruct((B, 1, A), jnp.float32),
    )(st2)
    return w3.reshape(B, A)


def _loss_body(w_ref, p_ref, t_ref, m_ref, iou_out, dist_out):
    i = pl.program_id(0)

    @pl.when(i == 0)
    def _init():
        iou_out[0, 0] = 0.0
        dist_out[0, 0] = 0.0

    w = w_ref[...]
    m = m_ref[...]
    iou, dist = _circle_losses(
        p_ref[:, 0, :], p_ref[:, 1, :], p_ref[:, 2, :],
        t_ref[:, 0, :], t_ref[:, 1, :], t_ref[:, 2, :])
    wm = w * m
    iou_out[0, 0] += jnp.sum((1.0 - iou) * wm)
    dist_out[0, 0] += jnp.sum((1.0 - dist) * wm)


@jax.jit
def _loss_sums(w, pt, tt, mt):
    return pl.pallas_call(
        _loss_body,
        grid=(GRID2,),
        in_specs=[
            pl.BlockSpec((B, AB2), lambda i: (0, i)),
            pl.BlockSpec((B, 3, AB2), lambda i: (0, 0, i)),
            pl.BlockSpec((B, 3, AB2), lambda i: (0, 0, i)),
            pl.BlockSpec((B, AB2), lambda i: (0, i)),
        ],
        out_specs=[
            pl.BlockSpec(memory_space=pltpu.SMEM),
            pl.BlockSpec(memory_space=pltpu.SMEM),
        ],
        out_shape=[
            jax.ShapeDtypeStruct((1, 1), jnp.float32),
            jax.ShapeDtypeStruct((1, 1), jnp.float32),
        ],
    )(w, pt, tt, mt)


def kernel(pred_dist, pred_bboxes, anchor_points, target_bboxes,
           target_scores, target_scores_sum, fg_mask):
    st = jnp.transpose(target_scores, (0, 2, 1))
    st2 = st.reshape(B * NC, A)
    pt = jnp.transpose(pred_bboxes, (0, 2, 1))
    tt = jnp.transpose(target_bboxes, (0, 2, 1))
    mt = fg_mask.astype(jnp.float32)
    w = _wsum(st2)
    si, sd = _loss_sums(w, pt, tt, mt)
    inv = 1.0 / target_scores_sum
    return (si[0, 0] * inv, sd[0, 0] * inv)


# speedup vs baseline: 1.2363x; 1.2363x over previous
import jax
import jax.numpy as jnp
from jax import lax
from jax.experimental import pallas as pl
from jax.experimental.pallas import tpu as pltpu

PI = 3.141592653589793
EPS = 1e-7

B, A, NC = 16, 21504, 80
ABLK = 2688
GRID = A // ABLK
NCH = NC // 2


def _acos(x):
    ax = jnp.abs(x)
    p = (1.5707963050 + ax * (-0.2145988016 + ax * (0.0889789874 + ax * (
        -0.0501743046 + ax * (0.0308918810 + ax * (-0.0170881256 + ax * (
            0.0066700901 + ax * -0.0012624911)))))))
    r = jnp.sqrt(jnp.maximum(1.0 - ax, 0.0)) * p
    return jnp.where(x >= 0.0, r, PI - r)


def _circle_losses(x1, y1, r1, x2, y2, r2):
    d2 = (x1 - x2) ** 2 + (y1 - y2) ** 2
    d = jnp.sqrt(jnp.maximum(d2, EPS))
    rsum = r1 + r2
    rdiff = jnp.abs(r1 - r2)
    rmin = jnp.minimum(r1, r2)
    no_overlap = d >= rsum
    contained = d <= rdiff
    a1 = jnp.clip((d2 + r1 ** 2 - r2 ** 2) / (2.0 * d * jnp.maximum(r1, EPS)),
                  -1.0 + 1e-6, 1.0 - 1e-6)
    a2 = jnp.clip((d2 + r2 ** 2 - r1 ** 2) / (2.0 * d * jnp.maximum(r2, EPS)),
                  -1.0 + 1e-6, 1.0 - 1e-6)
    tri = jnp.maximum((-d + rsum) * (d + r1 - r2) * (d - r1 + r2) * (d + rsum),
                      EPS)
    lens = (r1 ** 2 * _acos(a1) + r2 ** 2 * _acos(a2)
            - 0.5 * jnp.sqrt(tri))
    inter = jnp.where(no_overlap, 0.0, jnp.where(contained, PI * rmin ** 2, lens))
    union = PI * (r1 ** 2 + r2 ** 2) - inter
    iou = inter / (union + EPS)
    dist = jnp.clip(1.0 - d / (rsum + EPS), 0.0, 1.0)
    return iou, dist


def _loss_body(s1_ref, s2_ref, p_ref, t_ref, m_ref, iou_out, dist_out):
    i = pl.program_id(0)

    @pl.when(i == 0)
    def _init():
        iou_out[0, 0] = 0.0
        dist_out[0, 0] = 0.0

    w = jnp.sum(s1_ref[...], axis=1) + jnp.sum(s2_ref[...], axis=1)
    m = m_ref[...]
    iou, dist = _circle_losses(
        p_ref[:, 0, :], p_ref[:, 1, :], p_ref[:, 2, :],
        t_ref[:, 0, :], t_ref[:, 1, :], t_ref[:, 2, :])
    wm = w * m
    iou_out[0, 0] += jnp.sum((1.0 - iou) * wm)
    dist_out[0, 0] += jnp.sum((1.0 - dist) * wm)


@jax.jit
def _loss_sums(st, pt, tt, mt):
    return pl.pallas_call(
        _loss_body,
        grid=(GRID,),
        in_specs=[
            pl.BlockSpec((B, NCH, ABLK), lambda i: (0, 0, i)),
            pl.BlockSpec((B, NCH, ABLK), lambda i: (0, 1, i)),
            pl.BlockSpec((B, 3, ABLK), lambda i: (0, 0, i)),
            pl.BlockSpec((B, 3, ABLK), lambda i: (0, 0, i)),
            pl.BlockSpec((B, ABLK), lambda i: (0, i)),
        ],
        out_specs=[
            pl.BlockSpec(memory_space=pltpu.SMEM),
            pl.BlockSpec(memory_space=pltpu.SMEM),
        ],
        out_shape=[
            jax.ShapeDtypeStruct((1, 1), jnp.float32),
            jax.ShapeDtypeStruct((1, 1), jnp.float32),
        ],
    )(st, st, pt, tt, mt)


def kernel(pred_dist, pred_bboxes, anchor_points, target_bboxes,
           target_scores, target_scores_sum, fg_mask):
    st = jnp.transpose(target_scores, (0, 2, 1))
    pt = jnp.transpose(pred_bboxes, (0, 2, 1))
    tt = jnp.transpose(target_bboxes, (0, 2, 1))
    mt = fg_mask.astype(jnp.float32)
    si, sd = _loss_sums(st, pt, tt, mt)
    inv = 1.0 / target_scores_sum
    return (si[0, 0] * inv, sd[0, 0] * inv)
